# baseline (speedup 1.0000x reference)
import functools

import jax
import jax.numpy as jnp
from jax import lax
from jax.experimental import pallas as pl
from jax.experimental.pallas import tpu as pltpu
from jax.experimental.pallas import tpu_sc as plsc

N = 10000
E = 320000
D = 128

NC = 2
NS = 16
NW = NC * NS
NPAD = 10240
EPT = 10240
EPAD = EPT * NW
C = 80
NCHUNK = EPT // C
NPAIR = NCHUNK // 2
ROWS_PER_TILE = NPAD // NS
L = 16

_mesh = plsc.VectorSubcoreMesh(core_axis_name="c", subcore_axis_name="s")
_params = pltpu.CompilerParams(needs_layout_passes=False)


def _normalize_tc(x):
  def body(x_ref, xn_ref, n_ref):
    xv = x_ref[...]
    ss = jnp.sum(xv * xv, axis=1, keepdims=True)
    nn = jnp.sqrt(ss)
    inv = 1.0 / jnp.maximum(nn, 1e-30)
    xn_ref[:N] = xv * inv
    xn_ref[N:] = jnp.zeros((NPAD - N, D), jnp.float32)
    n_ref[:N] = nn
    n_ref[N:] = jnp.zeros((NPAD - N, 1), jnp.float32)
  return pl.pallas_call(
      body,
      out_shape=[
          jax.ShapeDtypeStruct((NPAD, D), jnp.float32),
          jax.ShapeDtypeStruct((NPAD, 1), jnp.float32),
      ],
  )(x)


def _combine_tc(partials):
  def body(p_ref, o_ref):
    o_ref[...] = p_ref[0, :N] + p_ref[1, :N]
  return pl.pallas_call(
      body,
      out_shape=jax.ShapeDtypeStruct((N, D), jnp.float32),
  )(partials)


def _pass_a(xn, nrm, row, col, betav):
  @functools.partial(
      pl.kernel,
      mesh=_mesh,
      compiler_params=_params,
      out_type=[
          jax.ShapeDtypeStruct((EPAD,), jnp.float32),
          jax.ShapeDtypeStruct((NPAD,), jnp.float32),
          jax.ShapeDtypeStruct((NPAD,), jnp.float32),
      ],
      scratch_types=[
          pltpu.VMEM((EPT,), jnp.int32),
          pltpu.VMEM((EPT,), jnp.int32),
          pltpu.VMEM((EPT,), jnp.float32),
          pltpu.VMEM((C, D), jnp.float32),
          pltpu.VMEM((C, D), jnp.float32),
          pltpu.VMEM((C, D), jnp.float32),
          pltpu.VMEM((C, D), jnp.float32),
          pltpu.VMEM((C,), jnp.float32),
          pltpu.VMEM((C,), jnp.float32),
          pltpu.VMEM((C,), jnp.int32),
          pltpu.VMEM((C,), jnp.int32),
          pltpu.VMEM((NPAD,), jnp.float32),
          pltpu.VMEM((L,), jnp.float32),
          pltpu.VMEM((NPAD,), jnp.float32),
          pltpu.VMEM_SHARED((NPAD,), jnp.float32),
          pltpu.SemaphoreType.DMA,
          pltpu.SemaphoreType.DMA,
          pltpu.SemaphoreType.DMA,
          pltpu.SemaphoreType.DMA,
          pltpu.SemaphoreType.DMA,
          pltpu.SemaphoreType.DMA,
      ],
  )
  def k2(xn_hbm, nrm_hbm, row_hbm, col_hbm, beta_hbm,
         w_hbm, denom0_hbm, denom1_hbm,
         rowfull_v, colfull_v, wfull_v,
         rows_r0, rows_c0, rows_r1, rows_c1,
         wbuf0, wbuf1, idxbuf0, idxbuf1,
         nrm_v, beta_v, zeros_v, denom_sh,
         sgr0, sgc0, sgr1, sgc1, sd0, sd1):
    cid = lax.axis_index("c")
    sid = lax.axis_index("s")
    wid = cid * NS + sid
    ebase = wid * EPT

    pltpu.sync_copy(nrm_hbm, nrm_v)
    pltpu.sync_copy(beta_hbm, beta_v)
    pltpu.sync_copy(row_hbm.at[pl.ds(ebase, EPT)], rowfull_v)
    pltpu.sync_copy(col_hbm.at[pl.ds(ebase, EPT)], colfull_v)

    def zbody(i, _):
      zeros_v[pl.ds(i * L, L)] = jnp.zeros((L,), jnp.float32)
      return 0
    lax.fori_loop(0, NPAD // L, zbody, 0)

    @pl.when(sid == 0)
    def _():
      pltpu.sync_copy(zeros_v, denom_sh)
    plsc.subcore_barrier()

    bvec = beta_v[...]
    lane_iota = lax.iota(jnp.int32, L)

    def gathers(ch, br, bc, sr, sc):
      sl = pl.ds(ch * C, C)
      return (pltpu.make_async_copy(xn_hbm.at[rowfull_v.at[sl]], br, sr),
              pltpu.make_async_copy(xn_hbm.at[colfull_v.at[sl]], bc, sc))

    def start_gathers(ch, br, bc, sr, sc):
      a, b = gathers(ch, br, bc, sr, sc)
      a.start()
      b.start()

    def wait_gathers(ch, br, bc, sr, sc):
      a, b = gathers(ch, br, bc, sr, sc)
      a.wait()
      b.wait()

    def compute_chunk(ch, rows_r, rows_c, wbuf, idxbuf):
      for g in range(C // L):
        off = ch * C + g * L
        ir = rowfull_v[pl.ds(off, L)]
        ic = colfull_v[pl.ds(off, L)]
        def ebody(e, res):
          ei = g * L + e
          acc = rows_r[ei, pl.ds(0, L)] * rows_c[ei, pl.ds(0, L)]
          for k in range(1, D // L):
            acc = acc + rows_r[ei, pl.ds(k * L, L)] * rows_c[ei, pl.ds(k * L, L)]
          return jnp.where(lane_iota == e, jnp.full((L,), jnp.sum(acc)), res)
        inner = lax.fori_loop(0, L, ebody, jnp.zeros((L,), jnp.float32))
        n_r = plsc.load_gather(nrm_v, [ir])
        n_c = plsc.load_gather(nrm_v, [ic])
        nprod = n_r * n_c
        w = jnp.exp(inner * (nprod / (nprod + 1e-7)) * bvec)
        wbuf[pl.ds(g * L, L)] = w
        idxbuf[pl.ds(g * L, L)] = ir
        wfull_v[pl.ds(off, L)] = w

    def denom_scatter(wbuf, idxbuf, sem):
      return pltpu.make_async_copy(wbuf, denom_sh.at[idxbuf], sem)

    start_gathers(0, rows_r0, rows_c0, sgr0, sgc0)
    start_gathers(1, rows_r1, rows_c1, sgr1, sgc1)

    def pair_body(k, _):
      ch0 = 2 * k
      @pl.when(k > 0)
      def _():
        denom_scatter(wbuf0, idxbuf0, sd0).wait()
      wait_gathers(ch0, rows_r0, rows_c0, sgr0, sgc0)
      compute_chunk(ch0, rows_r0, rows_c0, wbuf0, idxbuf0)
      pltpu.async_copy(wbuf0, denom_sh.at[idxbuf0], sd0, add=True)
      @pl.when(ch0 + 2 < NCHUNK)
      def _():
        start_gathers(ch0 + 2, rows_r0, rows_c0, sgr0, sgc0)

      ch1 = 2 * k + 1
      @pl.when(k > 0)
      def _():
        denom_scatter(wbuf1, idxbuf1, sd1).wait()
      wait_gathers(ch1, rows_r1, rows_c1, sgr1, sgc1)
      compute_chunk(ch1, rows_r1, rows_c1, wbuf1, idxbuf1)
      pltpu.async_copy(wbuf1, denom_sh.at[idxbuf1], sd1, add=True)
      @pl.when(ch1 + 2 < NCHUNK)
      def _():
        start_gathers(ch1 + 2, rows_r1, rows_c1, sgr1, sgc1)
      return 0

    lax.fori_loop(0, NPAIR, pair_body, 0)

    denom_scatter(wbuf0, idxbuf0, sd0).wait()
    denom_scatter(wbuf1, idxbuf1, sd1).wait()

    pltpu.sync_copy(wfull_v, w_hbm.at[pl.ds(ebase, EPT)])

    plsc.subcore_barrier()
    @pl.when(jnp.logical_and(sid == 0, cid == 0))
    def _():
      pltpu.sync_copy(denom_sh, denom0_hbm)
    @pl.when(jnp.logical_and(sid == 0, cid == 1))
    def _():
      pltpu.sync_copy(denom_sh, denom1_hbm)

  return k2(xn, nrm, row, col, betav)


def _pass_b(xn, nrm, row, col, w, denom0, denom1, znd):
  @functools.partial(
      pl.kernel,
      mesh=_mesh,
      compiler_params=_params,
      out_type=jax.ShapeDtypeStruct((NC, NPAD, D), jnp.float32),
      scratch_types=[
          pltpu.VMEM((C, D), jnp.float32),
          pltpu.VMEM((C, D), jnp.float32),
          pltpu.VMEM((C,), jnp.int32),
          pltpu.VMEM((C,), jnp.int32),
          pltpu.VMEM((C,), jnp.int32),
          pltpu.VMEM((C,), jnp.int32),
          pltpu.VMEM((C,), jnp.float32),
          pltpu.VMEM((C,), jnp.float32),
          pltpu.VMEM((L,), jnp.float32),
          pltpu.VMEM((NPAD,), jnp.float32),
          pltpu.VMEM((NPAD,), jnp.float32),
          pltpu.VMEM_SHARED((NPAD, D), jnp.float32),
          pltpu.SemaphoreType.DMA,
          pltpu.SemaphoreType.DMA,
          pltpu.SemaphoreType.DMA,
          pltpu.SemaphoreType.DMA,
          pltpu.SemaphoreType.DMA,
          pltpu.SemaphoreType.DMA,
          pltpu.SemaphoreType.DMA,
          pltpu.SemaphoreType.DMA,
          pltpu.SemaphoreType.DMA,
          pltpu.SemaphoreType.DMA,
      ],
  )
  def k3(xn_hbm, nrm_hbm, row_hbm, col_hbm, w_hbm, denom0_hbm, denom1_hbm,
         znd_hbm, out_hbm,
         rows0, rows1, idxr0, idxr1, idxc0, idxc1, wc0, wc1,
         svbuf, h_v, tmp_v, out_sh,
         sg0, sg1, ss0, ss1, sir0, sir1, sic0, sic1, swc0, swc1):
    cid = lax.axis_index("c")
    sid = lax.axis_index("s")
    wid = cid * NS + sid
    ebase = wid * EPT

    pltpu.sync_copy(denom0_hbm, h_v)
    pltpu.sync_copy(denom1_hbm, tmp_v)
    def hbody1(i, _):
      sl = pl.ds(i * L, L)
      h_v[sl] = h_v[sl] + tmp_v[sl] + 1e-16
      return 0
    lax.fori_loop(0, NPAD // L, hbody1, 0)
    pltpu.sync_copy(nrm_hbm, tmp_v)
    def hbody2(i, _):
      sl = pl.ds(i * L, L)
      h_v[sl] = tmp_v[sl] / h_v[sl]
      return 0
    lax.fori_loop(0, NPAD // L, hbody2, 0)

    sl_rows = pl.ds(sid * ROWS_PER_TILE, ROWS_PER_TILE)
    pltpu.sync_copy(znd_hbm.at[sl_rows], out_sh.at[sl_rows])
    plsc.subcore_barrier()

    def ld_idxr(ch, buf, sem):
      return pltpu.make_async_copy(
          row_hbm.at[pl.ds(ebase + ch * C, C)], buf, sem)
    def ld_idxc(ch, buf, sem):
      return pltpu.make_async_copy(
          col_hbm.at[pl.ds(ebase + ch * C, C)], buf, sem)
    def ld_wc(ch, buf, sem):
      return pltpu.make_async_copy(
          w_hbm.at[pl.ds(ebase + ch * C, C)], buf, sem)
    def gather(idxr, buf, sem):
      return pltpu.make_async_copy(xn_hbm.at[idxr], buf, sem)
    def scatter(buf, idxc, sem):
      return pltpu.make_async_copy(buf, out_sh.at[idxc], sem)

    def scale_chunk(rows, idxr, wc):
      for g in range(C // L):
        ir = idxr[pl.ds(g * L, L)]
        s_vec = wc[pl.ds(g * L, L)] * plsc.load_gather(h_v, [ir])
        svbuf[...] = s_vec
        def ebody(e, _):
          ei = g * L + e
          spv = plsc.load_gather(svbuf, [jnp.full((L,), e, jnp.int32)])
          for k in range(D // L):
            sl = pl.ds(k * L, L)
            rows[ei, sl] = rows[ei, sl] * spv
          return 0
        lax.fori_loop(0, L, ebody, 0)

    ld_idxr(0, idxr0, sir0).start()
    ld_idxc(0, idxc0, sic0).start()
    ld_wc(0, wc0, swc0).start()
    ld_idxr(1, idxr1, sir1).start()
    ld_idxc(1, idxc1, sic1).start()
    ld_wc(1, wc1, swc1).start()
    ld_idxr(0, idxr0, sir0).wait()
    gather(idxr0, rows0, sg0).start()
    ld_idxr(1, idxr1, sir1).wait()
    gather(idxr1, rows1, sg1).start()

    def half_body(ch, rows, idxr, idxc, wc, sg, ss, sir, sic, swc):
      gather(idxr, rows, sg).wait()
      ld_idxc(ch, idxc, sic).wait()
      ld_wc(ch, wc, swc).wait()
      scale_chunk(rows, idxr, wc)
      pltpu.async_copy(rows, out_sh.at[idxc], ss, add=True)
      @pl.when(ch + 2 < NCHUNK)
      def _():
        ld_idxr(ch + 2, idxr, sir).start()
        ld_wc(ch + 2, wc, swc).start()
      scatter(rows, idxc, ss).wait()
      @pl.when(ch + 2 < NCHUNK)
      def _():
        ld_idxc(ch + 2, idxc, sic).start()
        ld_idxr(ch + 2, idxr, sir).wait()
        gather(idxr, rows, sg).start()

    def pair_body(k, _):
      half_body(2 * k, rows0, idxr0, idxc0, wc0, sg0, ss0, sir0, sic0, swc0)
      half_body(2 * k + 1, rows1, idxr1, idxc1, wc1, sg1, ss1, sir1, sic1, swc1)
      return 0

    lax.fori_loop(0, NPAIR, pair_body, 0)

    plsc.subcore_barrier()
    pltpu.sync_copy(out_sh.at[sl_rows], out_hbm.at[cid, sl_rows])

  return k3(xn, nrm, row, col, w, denom0, denom1, znd)


def kernel(x, edge_index, beta):
  row = edge_index[0]
  col = edge_index[1]
  pad_ids = jnp.full((EPAD - E,), N, jnp.int32)
  rowp = jnp.concatenate([row, pad_ids])
  colp = jnp.concatenate([col, pad_ids])
  xn, n2 = _normalize_tc(x)
  nrm = n2.reshape(NPAD)
  betav = jnp.full((L,), beta, jnp.float32)
  w, denom0, denom1 = _pass_a(xn, nrm, rowp, colp, betav)
  znd = jnp.zeros((NPAD, D), jnp.float32)
  partials = _pass_b(xn, nrm, rowp, colp, w, denom0, denom1, znd)
  return _combine_tc(partials)


# speedup vs baseline: 3.9336x; 3.9336x over previous
import functools

import jax
import jax.numpy as jnp
from jax import lax
from jax.experimental import pallas as pl
from jax.experimental.pallas import tpu as pltpu
from jax.experimental.pallas import tpu_sc as plsc

N = 10000
E = 320000
D = 128

NC = 2
NS = 16
NW = NC * NS
NPAD = 10240
EPT = 10240
EPAD = EPT * NW
C = 80
NCHUNK = EPT // C
NPAIR = NCHUNK // 2
ROWS_PER_TILE = NPAD // NS
L = 16

_mesh = plsc.VectorSubcoreMesh(core_axis_name="c", subcore_axis_name="s")
_params = pltpu.CompilerParams(needs_layout_passes=False)


def _normalize_tc(x):
  def body(x_ref, xn_ref, n_ref):
    xv = x_ref[...]
    ss = jnp.sum(xv * xv, axis=1, keepdims=True)
    nn = jnp.sqrt(ss)
    inv = 1.0 / jnp.maximum(nn, 1e-30)
    xn_ref[:N] = xv * inv
    xn_ref[N:] = jnp.zeros((NPAD - N, D), jnp.float32)
    n_ref[:N] = nn
    n_ref[N:] = jnp.zeros((NPAD - N, 1), jnp.float32)
  return pl.pallas_call(
      body,
      out_shape=[
          jax.ShapeDtypeStruct((NPAD, D), jnp.float32),
          jax.ShapeDtypeStruct((NPAD, 1), jnp.float32),
      ],
  )(x)


def _combine_tc(partials):
  def body(p_ref, o_ref):
    o_ref[...] = p_ref[0, :N] + p_ref[1, :N]
  return pl.pallas_call(
      body,
      out_shape=jax.ShapeDtypeStruct((N, D), jnp.float32),
  )(partials)


def _pass_a(xn, nrm, row, col, betav):
  @functools.partial(
      pl.kernel,
      mesh=_mesh,
      compiler_params=_params,
      out_type=[
          jax.ShapeDtypeStruct((EPAD,), jnp.float32),
          jax.ShapeDtypeStruct((NPAD,), jnp.float32),
          jax.ShapeDtypeStruct((NPAD,), jnp.float32),
      ],
      scratch_types=[
          pltpu.VMEM((EPT,), jnp.int32),
          pltpu.VMEM((EPT,), jnp.int32),
          pltpu.VMEM((EPT,), jnp.float32),
          pltpu.VMEM((C, D), jnp.float32),
          pltpu.VMEM((C, D), jnp.float32),
          pltpu.VMEM((C, D), jnp.float32),
          pltpu.VMEM((C, D), jnp.float32),
          pltpu.VMEM((C,), jnp.float32),
          pltpu.VMEM((C,), jnp.float32),
          pltpu.VMEM((C,), jnp.int32),
          pltpu.VMEM((C,), jnp.int32),
          pltpu.VMEM((NPAD,), jnp.float32),
          pltpu.VMEM((L,), jnp.float32),
          pltpu.VMEM((NPAD,), jnp.float32),
          pltpu.VMEM_SHARED((NPAD,), jnp.float32),
          pltpu.SemaphoreType.DMA,
          pltpu.SemaphoreType.DMA,
          pltpu.SemaphoreType.DMA,
          pltpu.SemaphoreType.DMA,
          pltpu.SemaphoreType.DMA,
          pltpu.SemaphoreType.DMA,
      ],
  )
  def k2(xn_hbm, nrm_hbm, row_hbm, col_hbm, beta_hbm,
         w_hbm, denom0_hbm, denom1_hbm,
         rowfull_v, colfull_v, wfull_v,
         rows_r0, rows_c0, rows_r1, rows_c1,
         wbuf0, wbuf1, idxbuf0, idxbuf1,
         nrm_v, beta_v, zeros_v, denom_sh,
         sgr0, sgc0, sgr1, sgc1, sd0, sd1):
    cid = lax.axis_index("c")
    sid = lax.axis_index("s")
    wid = cid * NS + sid
    ebase = wid * EPT

    pltpu.sync_copy(nrm_hbm, nrm_v)
    pltpu.sync_copy(beta_hbm, beta_v)
    pltpu.sync_copy(row_hbm.at[pl.ds(ebase, EPT)], rowfull_v)
    pltpu.sync_copy(col_hbm.at[pl.ds(ebase, EPT)], colfull_v)

    def zbody(i, _):
      zeros_v[pl.ds(i * L, L)] = jnp.zeros((L,), jnp.float32)
      return 0
    lax.fori_loop(0, NPAD // L, zbody, 0)

    @pl.when(sid == 0)
    def _():
      pltpu.sync_copy(zeros_v, denom_sh)
    plsc.subcore_barrier()

    bvec = beta_v[...]
    lane_iota = lax.iota(jnp.int32, L)

    def gathers(ch, br, bc, sr, sc):
      sl = pl.ds(ch * C, C)
      return (pltpu.make_async_copy(xn_hbm.at[rowfull_v.at[sl]], br, sr),
              pltpu.make_async_copy(xn_hbm.at[colfull_v.at[sl]], bc, sc))

    def start_gathers(ch, br, bc, sr, sc):
      a, b = gathers(ch, br, bc, sr, sc)
      a.start()
      b.start()

    def wait_gathers(ch, br, bc, sr, sc):
      a, b = gathers(ch, br, bc, sr, sc)
      a.wait()
      b.wait()

    def compute_chunk(ch, rows_r, rows_c, wbuf, idxbuf):
      for g in range(C // L):
        off = ch * C + g * L
        ir = rowfull_v[pl.ds(off, L)]
        ic = colfull_v[pl.ds(off, L)]
        def ebody(e, res):
          ei = g * L + e
          acc = rows_r[ei, pl.ds(0, L)] * rows_c[ei, pl.ds(0, L)]
          for k in range(1, D // L):
            acc = acc + rows_r[ei, pl.ds(k * L, L)] * rows_c[ei, pl.ds(k * L, L)]
          return jnp.where(lane_iota == e, jnp.full((L,), jnp.sum(acc)), res)
        inner = lax.fori_loop(0, L, ebody, jnp.zeros((L,), jnp.float32))
        n_r = plsc.load_gather(nrm_v, [ir])
        n_c = plsc.load_gather(nrm_v, [ic])
        nprod = n_r * n_c
        w = jnp.exp(inner * (nprod / (nprod + 1e-7)) * bvec)
        wbuf[pl.ds(g * L, L)] = w
        idxbuf[pl.ds(g * L, L)] = ir
        wfull_v[pl.ds(off, L)] = w

    def denom_scatter(wbuf, idxbuf, sem):
      return pltpu.make_async_copy(wbuf, denom_sh.at[idxbuf], sem)

    start_gathers(0, rows_r0, rows_c0, sgr0, sgc0)
    start_gathers(1, rows_r1, rows_c1, sgr1, sgc1)

    def pair_body(k, _):
      ch0 = 2 * k
      @pl.when(k > 0)
      def _():
        denom_scatter(wbuf0, idxbuf0, sd0).wait()
      wait_gathers(ch0, rows_r0, rows_c0, sgr0, sgc0)
      compute_chunk(ch0, rows_r0, rows_c0, wbuf0, idxbuf0)
      pltpu.async_copy(wbuf0, denom_sh.at[idxbuf0], sd0, add=True)
      @pl.when(ch0 + 2 < NCHUNK)
      def _():
        start_gathers(ch0 + 2, rows_r0, rows_c0, sgr0, sgc0)

      ch1 = 2 * k + 1
      @pl.when(k > 0)
      def _():
        denom_scatter(wbuf1, idxbuf1, sd1).wait()
      wait_gathers(ch1, rows_r1, rows_c1, sgr1, sgc1)
      compute_chunk(ch1, rows_r1, rows_c1, wbuf1, idxbuf1)
      pltpu.async_copy(wbuf1, denom_sh.at[idxbuf1], sd1, add=True)
      @pl.when(ch1 + 2 < NCHUNK)
      def _():
        start_gathers(ch1 + 2, rows_r1, rows_c1, sgr1, sgc1)
      return 0

    lax.fori_loop(0, NPAIR, pair_body, 0)

    denom_scatter(wbuf0, idxbuf0, sd0).wait()
    denom_scatter(wbuf1, idxbuf1, sd1).wait()

    pltpu.sync_copy(wfull_v, w_hbm.at[pl.ds(ebase, EPT)])

    plsc.subcore_barrier()
    @pl.when(jnp.logical_and(sid == 0, cid == 0))
    def _():
      pltpu.sync_copy(denom_sh, denom0_hbm)
    @pl.when(jnp.logical_and(sid == 0, cid == 1))
    def _():
      pltpu.sync_copy(denom_sh, denom1_hbm)

  return k2(xn, nrm, row, col, betav)


def _pass_b(xn, nrm, row, col, w, denom0, denom1, znd):
  @functools.partial(
      pl.kernel,
      mesh=_mesh,
      compiler_params=_params,
      out_type=jax.ShapeDtypeStruct((NC, NPAD, D), jnp.float32),
      scratch_types=[
          pltpu.VMEM((C, D), jnp.float32),
          pltpu.VMEM((C, D), jnp.float32),
          pltpu.VMEM((C,), jnp.int32),
          pltpu.VMEM((C,), jnp.int32),
          pltpu.VMEM((C,), jnp.int32),
          pltpu.VMEM((C,), jnp.int32),
          pltpu.VMEM((C,), jnp.float32),
          pltpu.VMEM((C,), jnp.float32),
          pltpu.VMEM((L,), jnp.float32),
          pltpu.VMEM((NPAD,), jnp.float32),
          pltpu.VMEM((NPAD,), jnp.float32),
          pltpu.VMEM_SHARED((NPAD, D), jnp.float32),
          pltpu.SemaphoreType.DMA,
          pltpu.SemaphoreType.DMA,
          pltpu.SemaphoreType.DMA,
          pltpu.SemaphoreType.DMA,
          pltpu.SemaphoreType.DMA,
          pltpu.SemaphoreType.DMA,
          pltpu.SemaphoreType.DMA,
          pltpu.SemaphoreType.DMA,
          pltpu.SemaphoreType.DMA,
          pltpu.SemaphoreType.DMA,
      ],
  )
  def k3(xn_hbm, nrm_hbm, row_hbm, col_hbm, w_hbm, denom0_hbm, denom1_hbm,
         znd_hbm, out_hbm,
         rows0, rows1, idxr0, idxr1, idxc0, idxc1, wc0, wc1,
         svbuf, h_v, tmp_v, out_sh,
         sg0, sg1, ss0, ss1, sir0, sir1, sic0, sic1, swc0, swc1):
    cid = lax.axis_index("c")
    sid = lax.axis_index("s")
    wid = cid * NS + sid
    ebase = wid * EPT

    pltpu.sync_copy(denom0_hbm, h_v)
    pltpu.sync_copy(denom1_hbm, tmp_v)
    def hbody1(i, _):
      sl = pl.ds(i * L, L)
      h_v[sl] = h_v[sl] + tmp_v[sl] + 1e-16
      return 0
    lax.fori_loop(0, NPAD // L, hbody1, 0)
    pltpu.sync_copy(nrm_hbm, tmp_v)
    def hbody2(i, _):
      sl = pl.ds(i * L, L)
      h_v[sl] = tmp_v[sl] / h_v[sl]
      return 0
    lax.fori_loop(0, NPAD // L, hbody2, 0)

    sl_rows = pl.ds(sid * ROWS_PER_TILE, ROWS_PER_TILE)
    pltpu.sync_copy(znd_hbm.at[sl_rows], out_sh.at[sl_rows])
    plsc.subcore_barrier()

    def ld_idxr(ch, buf, sem):
      return pltpu.make_async_copy(
          row_hbm.at[pl.ds(ebase + ch * C, C)], buf, sem)
    def ld_idxc(ch, buf, sem):
      return pltpu.make_async_copy(
          col_hbm.at[pl.ds(ebase + ch * C, C)], buf, sem)
    def ld_wc(ch, buf, sem):
      return pltpu.make_async_copy(
          w_hbm.at[pl.ds(ebase + ch * C, C)], buf, sem)
    def gather(idxr, buf, sem):
      return pltpu.make_async_copy(xn_hbm.at[idxr], buf, sem)
    def scatter(buf, idxc, sem):
      return pltpu.make_async_copy(buf, out_sh.at[idxc], sem)

    def scale_chunk(rows, idxr, wc):
      for g in range(C // L):
        ir = idxr[pl.ds(g * L, L)]
        s_vec = wc[pl.ds(g * L, L)] * plsc.load_gather(h_v, [ir])
        svbuf[...] = s_vec
        def ebody(e, _):
          ei = g * L + e
          spv = plsc.load_gather(svbuf, [jnp.full((L,), e, jnp.int32)])
          for k in range(D // L):
            sl = pl.ds(k * L, L)
            rows[ei, sl] = rows[ei, sl] * spv
          return 0
        lax.fori_loop(0, L, ebody, 0)

    ld_idxr(0, idxr0, sir0).start()
    ld_idxc(0, idxc0, sic0).start()
    ld_wc(0, wc0, swc0).start()
    ld_idxr(1, idxr1, sir1).start()
    ld_idxc(1, idxc1, sic1).start()
    ld_wc(1, wc1, swc1).start()
    ld_idxr(0, idxr0, sir0).wait()
    gather(idxr0, rows0, sg0).start()
    ld_idxr(1, idxr1, sir1).wait()
    gather(idxr1, rows1, sg1).start()

    def half_body(ch, rows, idxr, idxc, wc, sg, ss, sir, sic, swc):
      gather(idxr, rows, sg).wait()
      ld_idxc(ch, idxc, sic).wait()
      ld_wc(ch, wc, swc).wait()
      scale_chunk(rows, idxr, wc)
      pltpu.async_copy(rows, out_sh.at[idxc], ss, add=True)
      @pl.when(ch + 2 < NCHUNK)
      def _():
        ld_idxr(ch + 2, idxr, sir).start()
        ld_wc(ch + 2, wc, swc).start()
      scatter(rows, idxc, ss).wait()
      @pl.when(ch + 2 < NCHUNK)
      def _():
        ld_idxc(ch + 2, idxc, sic).start()
        ld_idxr(ch + 2, idxr, sir).wait()
        gather(idxr, rows, sg).start()

    def pair_body(k, _):
      half_body(2 * k, rows0, idxr0, idxc0, wc0, sg0, ss0, sir0, sic0, swc0)
      half_body(2 * k + 1, rows1, idxr1, idxc1, wc1, sg1, ss1, sir1, sic1, swc1)
      return 0

    lax.fori_loop(0, NPAIR, pair_body, 0)

    plsc.subcore_barrier()
    pltpu.sync_copy(out_sh.at[sl_rows], out_hbm.at[cid, sl_rows])

  return k3(xn, nrm, row, col, w, denom0, denom1, znd)


def kernel(x, edge_index, beta):
  row = edge_index[0]
  col = edge_index[1]
  pad_ids = N + (jnp.arange(EPAD - E, dtype=jnp.int32) % (NPAD - N))
  rowp = jnp.concatenate([row, pad_ids])
  colp = jnp.concatenate([col, pad_ids])
  xn, n2 = _normalize_tc(x)
  nrm = n2.reshape(NPAD)
  betav = jnp.full((L,), beta, jnp.float32)
  w, denom0, denom1 = _pass_a(xn, nrm, rowp, colp, betav)
  znd = jnp.zeros((NPAD, D), jnp.float32)
  partials = _pass_b(xn, nrm, rowp, colp, w, denom0, denom1, znd)
  return _combine_tc(partials)
